# baseline (device time: 38147 ns/iter reference)
import os

import jax
import jax.numpy as jnp
from jax import lax
from jax.experimental import pallas as pl
from jax.experimental.pallas import tpu as pltpu

_SKIP_COMM = os.environ.get("SKIP_COMM", "0") == "1"

N_DEV = 4
B_LOC = 2
SQ = 256
SKV = 256
HQ = 16
DH = 64
D_MODEL = 512
D_HEADS = HQ * DH
CHUNK = D_HEADS // N_DEV
H_PER = HQ // N_DEV
HALF_Q = D_MODEL // 2
HALF_O = CHUNK // 2
BLK = 64


def kernel(x, Wq, K_ext, V_ext, Wo):
    k2 = K_ext.reshape(8, SKV, D_HEADS)
    v2 = V_ext.reshape(8, SKV, D_HEADS)

    def body(x_ref, wq_ref, k_ref, v_ref, wo_ref, out_ref,
             commq, commo, kscr, vscr,
             sendq, recvq, sendo, recvo, kvsem):
        my = lax.axis_index("i")
        left = lax.rem(my + N_DEV - 1, N_DEV)
        right = lax.rem(my + 1, N_DEV)
        opp = lax.rem(my + 2, N_DEV)
        origin_of_slot = [my, left, right, opp]

        barrier = pltpu.get_barrier_semaphore()
        for nbr in (left, right):
            pl.semaphore_signal(barrier, inc=1, device_id=(nbr,),
                                device_id_type=pl.DeviceIdType.MESH)
        pl.semaphore_wait(barrier, 2)

        commq[0] = wq_ref[...].astype(jnp.bfloat16)
        commo[0] = wo_ref[...].astype(jnp.bfloat16)

        def copies(src, dst, dev, idx):
            cq = pltpu.make_async_remote_copy(
                src_ref=src(commq), dst_ref=dst(commq),
                send_sem=sendq.at[idx], recv_sem=recvq.at[idx],
                device_id=(dev,), device_id_type=pl.DeviceIdType.MESH)
            co = pltpu.make_async_remote_copy(
                src_ref=src(commo), dst_ref=dst(commo),
                send_sem=sendo.at[idx], recv_sem=recvo.at[idx],
                device_id=(dev,), device_id_type=pl.DeviceIdType.MESH)
            return cq, co

        aL = copies(lambda c: c.at[0], lambda c: c.at[2], left, 0)
        aR = copies(lambda c: c.at[0], lambda c: c.at[1], right, 1)
        bR = copies(lambda c: c.at[1, 0:(HALF_Q if c is commq else HALF_O)],
                    lambda c: c.at[3, 0:(HALF_Q if c is commq else HALF_O)],
                    right, 2)
        bL = copies(lambda c: c.at[2, (HALF_Q if c is commq else HALF_O):],
                    lambda c: c.at[3, (HALF_Q if c is commq else HALF_O):],
                    left, 3)

        if not _SKIP_COMM:
            for c in aL + aR:
                c.start()

        kv_dma = []
        for s in range(N_DEV):
            off = origin_of_slot[s] * CHUNK
            per_slot = []
            for b in range(B_LOC):
                gb = my * B_LOC + b
                for t, (src, dst) in enumerate(((k_ref, kscr), (v_ref, vscr))):
                    dma = pltpu.make_async_copy(
                        src.at[gb, :, pl.ds(off, CHUNK)],
                        dst.at[b, s],
                        kvsem.at[s, b, t])
                    dma.start()
                    per_slot.append(dma)
            kv_dma.append(per_slot)

        r2 = lax.broadcasted_iota(jnp.int32, (CHUNK, DH), 0)
        c2 = lax.broadcasted_iota(jnp.int32, (CHUNK, DH), 1)
        sel = [(r2 == c2 + j * DH).astype(jnp.bfloat16) for j in range(H_PER)]

        xb = [x_ref[b].astype(jnp.bfloat16) for b in range(B_LOC)]

        def attend(qh, kh, vh):
            q_o = jnp.concatenate([qh[0:BLK], qh[3 * BLK:]], axis=0)
            k_o = jnp.concatenate([kh[0:BLK], kh[3 * BLK:]], axis=0)
            v_o = jnp.concatenate([vh[0:BLK], vh[3 * BLK:]], axis=0)
            s_o = lax.dot_general(
                q_o, k_o, (((1,), (1,)), ((), ())),
                preferred_element_type=jnp.float32) * 0.125
            w_o = jnp.exp(s_o)
            r_o = 1.0 / jnp.sum(w_o, axis=1, keepdims=True)
            c_o = jnp.dot(w_o.astype(jnp.bfloat16), v_o,
                          preferred_element_type=jnp.float32) * r_o

            q_m = qh[BLK:3 * BLK]
            s_m = lax.dot_general(
                q_m, kh[0:3 * BLK], (((1,), (1,)), ((), ())),
                preferred_element_type=jnp.float32) * 0.125
            w_m = jnp.exp(s_m)
            r_m = 1.0 / jnp.sum(w_m, axis=1, keepdims=True)
            c_m = jnp.dot(w_m.astype(jnp.bfloat16), vh[0:3 * BLK],
                          preferred_element_type=jnp.float32) * r_m
            return jnp.concatenate([c_o[0:BLK], c_m, c_o[BLK:]], axis=0)

        def compute_chunk(slot):
            for dma in kv_dma[slot]:
                dma.wait()
            wq_c = commq[slot]
            wo_c = commo[slot]
            for b in range(B_LOC):
                qc = jnp.dot(xb[b], wq_c,
                             preferred_element_type=jnp.float32
                             ).astype(jnp.bfloat16)
                kc = kscr[b, slot].astype(jnp.bfloat16)
                vc = vscr[b, slot].astype(jnp.bfloat16)
                acc = None
                for j in range(H_PER):
                    qh = jnp.dot(qc, sel[j],
                                 preferred_element_type=jnp.float32
                                 ).astype(jnp.bfloat16)
                    kh = jnp.dot(kc, sel[j],
                                 preferred_element_type=jnp.float32
                                 ).astype(jnp.bfloat16)
                    vh = jnp.dot(vc, sel[j],
                                 preferred_element_type=jnp.float32
                                 ).astype(jnp.bfloat16)
                    ctx_j = attend(qh, kh, vh).astype(jnp.bfloat16)
                    contrib = jnp.dot(ctx_j, wo_c[j * DH:(j + 1) * DH],
                                      preferred_element_type=jnp.float32)
                    acc = contrib if acc is None else acc + contrib
                if slot == 0:
                    out_ref[b] = acc
                else:
                    out_ref[b] = out_ref[b] + acc

        compute_chunk(0)
        if not _SKIP_COMM:
            aR[0].wait_recv()
            aR[1].wait_recv()
            bR[0].start()
            bR[1].start()
            aL[0].wait_recv()
            aL[1].wait_recv()
            bL[0].start()
            bL[1].start()
        compute_chunk(1)
        compute_chunk(2)
        if not _SKIP_COMM:
            for c in bR + bL:
                c.wait_recv()
        compute_chunk(3)
        if not _SKIP_COMM:
            for c in aL + aR + bR + bL:
                c.wait_send()

    return pl.pallas_call(
        body,
        out_shape=jax.ShapeDtypeStruct((B_LOC, SQ, D_MODEL), jnp.float32),
        in_specs=[
            pl.BlockSpec(memory_space=pltpu.VMEM),
            pl.BlockSpec(memory_space=pltpu.VMEM),
            pl.BlockSpec(memory_space=pltpu.MemorySpace.HBM),
            pl.BlockSpec(memory_space=pltpu.MemorySpace.HBM),
            pl.BlockSpec(memory_space=pltpu.VMEM),
        ],
        out_specs=pl.BlockSpec(memory_space=pltpu.VMEM),
        scratch_shapes=[
            pltpu.VMEM((N_DEV, D_MODEL, CHUNK), jnp.bfloat16),
            pltpu.VMEM((N_DEV, CHUNK, D_MODEL), jnp.bfloat16),
            pltpu.VMEM((B_LOC, N_DEV, SKV, CHUNK), jnp.float32),
            pltpu.VMEM((B_LOC, N_DEV, SKV, CHUNK), jnp.float32),
            pltpu.SemaphoreType.DMA((4,)),
            pltpu.SemaphoreType.DMA((4,)),
            pltpu.SemaphoreType.DMA((4,)),
            pltpu.SemaphoreType.DMA((4,)),
            pltpu.SemaphoreType.DMA((N_DEV, B_LOC, 2)),
        ],
        compiler_params=pltpu.CompilerParams(collective_id=0),
    )(x, Wq, k2, v2, Wo)


# device time: 30501 ns/iter; 1.2507x vs baseline; 1.2507x over previous
import os

import jax
import jax.numpy as jnp
from jax import lax
from jax.experimental import pallas as pl
from jax.experimental.pallas import tpu as pltpu

_SKIP_COMM = os.environ.get("SKIP_COMM", "0") == "1"

N_DEV = 4
B_LOC = 2
SQ = 256
SKV = 256
HQ = 16
DH = 64
D_MODEL = 512
D_HEADS = HQ * DH
CHUNK = D_HEADS // N_DEV
H_PER = HQ // N_DEV
HALF_Q = D_MODEL // 2
HALF_O = CHUNK // 2
BLK = 64


def kernel(x, Wq, K_ext, V_ext, Wo):
    k2 = K_ext.reshape(8, SKV, D_HEADS)
    v2 = V_ext.reshape(8, SKV, D_HEADS)

    def body(x_ref, wq_ref, k_ref, v_ref, wo_ref, out_ref,
             commq, commo, kscr, vscr,
             sendq, recvq, sendo, recvo, kvsem):
        my = lax.axis_index("i")
        left = lax.rem(my + N_DEV - 1, N_DEV)
        right = lax.rem(my + 1, N_DEV)
        opp = lax.rem(my + 2, N_DEV)
        origin_of_slot = [my, left, right, opp]

        barrier = pltpu.get_barrier_semaphore()
        for nbr in (left, right):
            pl.semaphore_signal(barrier, inc=1, device_id=(nbr,),
                                device_id_type=pl.DeviceIdType.MESH)
        pl.semaphore_wait(barrier, 2)

        commq[0] = wq_ref[...].astype(jnp.bfloat16)
        commo[0] = wo_ref[...].astype(jnp.bfloat16)

        def copies(src, dst, dev, idx):
            cq = pltpu.make_async_remote_copy(
                src_ref=src(commq), dst_ref=dst(commq),
                send_sem=sendq.at[idx], recv_sem=recvq.at[idx],
                device_id=(dev,), device_id_type=pl.DeviceIdType.MESH)
            co = pltpu.make_async_remote_copy(
                src_ref=src(commo), dst_ref=dst(commo),
                send_sem=sendo.at[idx], recv_sem=recvo.at[idx],
                device_id=(dev,), device_id_type=pl.DeviceIdType.MESH)
            return cq, co

        aL = copies(lambda c: c.at[0], lambda c: c.at[2], left, 0)
        aR = copies(lambda c: c.at[0], lambda c: c.at[1], right, 1)
        bR = copies(lambda c: c.at[1, 0:(HALF_Q if c is commq else HALF_O)],
                    lambda c: c.at[3, 0:(HALF_Q if c is commq else HALF_O)],
                    right, 2)
        bL = copies(lambda c: c.at[2, (HALF_Q if c is commq else HALF_O):],
                    lambda c: c.at[3, (HALF_Q if c is commq else HALF_O):],
                    left, 3)

        if not _SKIP_COMM:
            for c in aL + aR:
                c.start()

        kv_dma = []
        for s in range(N_DEV):
            off = origin_of_slot[s] * CHUNK
            per_slot = []
            for b in range(B_LOC):
                gb = my * B_LOC + b
                for t, (src, dst) in enumerate(((k_ref, kscr), (v_ref, vscr))):
                    dma = pltpu.make_async_copy(
                        src.at[gb, :, pl.ds(off, CHUNK)],
                        dst.at[b, s],
                        kvsem.at[s, b, t])
                    dma.start()
                    per_slot.append(dma)
            kv_dma.append(per_slot)

        ri = lax.broadcasted_iota(jnp.int32, (SQ, SKV), 0) // BLK
        ci = lax.broadcasted_iota(jnp.int32, (SQ, SKV), 1) // BLK
        mask = (ri == ci) | (ci == 0) | (lax.rem(ri + ci, 3) == 0)
        mask_cat = jnp.tile(mask, (1, H_PER))
        eye_r = lax.broadcasted_iota(jnp.int32, (SKV, SKV), 0)
        eye_c = lax.broadcasted_iota(jnp.int32, (SKV, SKV), 1)
        eye = (eye_r == eye_c).astype(jnp.bfloat16)
        sum_r = lax.broadcasted_iota(jnp.int32, (H_PER * SKV, H_PER), 0)
        sum_c = lax.broadcasted_iota(jnp.int32, (H_PER * SKV, H_PER), 1)
        ones_bd = (sum_r // SKV == sum_c).astype(jnp.bfloat16)
        ex_r = lax.broadcasted_iota(jnp.int32, (H_PER, CHUNK), 0)
        ex_c = lax.broadcasted_iota(jnp.int32, (H_PER, CHUNK), 1)
        expand = (ex_c // DH == ex_r).astype(jnp.bfloat16)
        colmask = [(lax.broadcasted_iota(jnp.int32, (SKV, CHUNK), 1) // DH
                    == j).astype(jnp.bfloat16) for j in range(H_PER)]

        xb = [x_ref[b].astype(jnp.bfloat16) for b in range(B_LOC)]

        def compute_chunk(slot):
            for dma in kv_dma[slot]:
                dma.wait()
            wq_c = commq[slot]
            wo_c = commo[slot]
            for b in range(B_LOC):
                qc = jnp.dot(xb[b], wq_c,
                             preferred_element_type=jnp.float32
                             ).astype(jnp.bfloat16)
                kc = kscr[b, slot].astype(jnp.bfloat16)
                vc = vscr[b, slot].astype(jnp.bfloat16)
                kcT = lax.dot_general(
                    kc, eye, (((0,), (0,)), ((), ())),
                    preferred_element_type=jnp.float32
                    ).astype(jnp.bfloat16)
                rows = []
                for j in range(H_PER):
                    pieces = []
                    if j > 0:
                        pieces.append(jnp.zeros((DH, j * SKV), jnp.bfloat16))
                    pieces.append(kcT[j * DH:(j + 1) * DH])
                    if j < H_PER - 1:
                        pieces.append(jnp.zeros(
                            (DH, (H_PER - 1 - j) * SKV), jnp.bfloat16))
                    rows.append(jnp.concatenate(pieces, axis=1))
                k_bd = jnp.concatenate(rows, axis=0)
                s_cat = jnp.dot(qc, k_bd,
                                preferred_element_type=jnp.float32)
                w_cat = jnp.where(mask_cat,
                                  jnp.exp(s_cat * 0.125),
                                  0.0).astype(jnp.bfloat16)
                sums = jnp.dot(w_cat, ones_bd,
                               preferred_element_type=jnp.float32)
                recip = (1.0 / sums).astype(jnp.bfloat16)
                scale = jnp.dot(recip, expand,
                                preferred_element_type=jnp.float32)
                v_bd = jnp.concatenate(
                    [vc * colmask[j] for j in range(H_PER)],
                    axis=0)
                ctx = jnp.dot(w_cat, v_bd,
                              preferred_element_type=jnp.float32)
                ctx = (ctx * scale).astype(jnp.bfloat16)
                acc = jnp.dot(ctx, wo_c,
                              preferred_element_type=jnp.float32)
                if slot == 0:
                    out_ref[b] = acc
                else:
                    out_ref[b] = out_ref[b] + acc

        compute_chunk(0)
        if not _SKIP_COMM:
            aR[0].wait_recv()
            aR[1].wait_recv()
            bR[0].start()
            bR[1].start()
            aL[0].wait_recv()
            aL[1].wait_recv()
            bL[0].start()
            bL[1].start()
        compute_chunk(1)
        compute_chunk(2)
        if not _SKIP_COMM:
            for c in bR + bL:
                c.wait_recv()
        compute_chunk(3)
        if not _SKIP_COMM:
            for c in aL + aR + bR + bL:
                c.wait_send()

    return pl.pallas_call(
        body,
        out_shape=jax.ShapeDtypeStruct((B_LOC, SQ, D_MODEL), jnp.float32),
        in_specs=[
            pl.BlockSpec(memory_space=pltpu.VMEM),
            pl.BlockSpec(memory_space=pltpu.VMEM),
            pl.BlockSpec(memory_space=pltpu.MemorySpace.HBM),
            pl.BlockSpec(memory_space=pltpu.MemorySpace.HBM),
            pl.BlockSpec(memory_space=pltpu.VMEM),
        ],
        out_specs=pl.BlockSpec(memory_space=pltpu.VMEM),
        scratch_shapes=[
            pltpu.VMEM((N_DEV, D_MODEL, CHUNK), jnp.bfloat16),
            pltpu.VMEM((N_DEV, CHUNK, D_MODEL), jnp.bfloat16),
            pltpu.VMEM((B_LOC, N_DEV, SKV, CHUNK), jnp.float32),
            pltpu.VMEM((B_LOC, N_DEV, SKV, CHUNK), jnp.float32),
            pltpu.SemaphoreType.DMA((4,)),
            pltpu.SemaphoreType.DMA((4,)),
            pltpu.SemaphoreType.DMA((4,)),
            pltpu.SemaphoreType.DMA((4,)),
            pltpu.SemaphoreType.DMA((N_DEV, B_LOC, 2)),
        ],
        compiler_params=pltpu.CompilerParams(collective_id=0),
    )(x, Wq, k2, v2, Wo)


# device time: 26105 ns/iter; 1.4613x vs baseline; 1.1684x over previous
import os

import jax
import jax.numpy as jnp
from jax import lax
from jax.experimental import pallas as pl
from jax.experimental.pallas import tpu as pltpu

_SKIP_COMM = os.environ.get("SKIP_COMM", "0") == "1"

N_DEV = 4
B_LOC = 2
SQ = 256
SKV = 256
HQ = 16
DH = 64
D_MODEL = 512
D_HEADS = HQ * DH
CHUNK = D_HEADS // N_DEV
H_PER = HQ // N_DEV
HALF_Q = D_MODEL // 2
HALF_O = CHUNK // 2
BLK = 64
WCAT = H_PER * SKV


def kernel(x, Wq, K_ext, V_ext, Wo):
    gb0 = lax.axis_index("i") * B_LOC
    k_loc = jnp.transpose(
        lax.dynamic_slice_in_dim(K_ext, gb0, B_LOC, axis=0)
        .astype(jnp.bfloat16), (0, 2, 1, 3))
    v_loc = jnp.transpose(
        lax.dynamic_slice_in_dim(V_ext, gb0, B_LOC, axis=0)
        .astype(jnp.bfloat16), (0, 2, 1, 3))

    def body(x_ref, wq_ref, k_ref, v_ref, wo_ref, out_ref,
             commq, commo, kbd,
             sendq, recvq, sendo, recvo):
        my = lax.axis_index("i")
        left = lax.rem(my + N_DEV - 1, N_DEV)
        right = lax.rem(my + 1, N_DEV)
        opp = lax.rem(my + 2, N_DEV)
        origin_of_slot = [my, left, right, opp]

        barrier = pltpu.get_barrier_semaphore()
        for nbr in (left, right):
            pl.semaphore_signal(barrier, inc=1, device_id=(nbr,),
                                device_id_type=pl.DeviceIdType.MESH)
        pl.semaphore_wait(barrier, 2)

        commq[0] = wq_ref[...].astype(jnp.bfloat16)
        commo[0] = wo_ref[...].astype(jnp.bfloat16)

        def copies(src, dst, dev, idx):
            cq = pltpu.make_async_remote_copy(
                src_ref=src(commq), dst_ref=dst(commq),
                send_sem=sendq.at[idx], recv_sem=recvq.at[idx],
                device_id=(dev,), device_id_type=pl.DeviceIdType.MESH)
            co = pltpu.make_async_remote_copy(
                src_ref=src(commo), dst_ref=dst(commo),
                send_sem=sendo.at[idx], recv_sem=recvo.at[idx],
                device_id=(dev,), device_id_type=pl.DeviceIdType.MESH)
            return cq, co

        aL = copies(lambda c: c.at[0], lambda c: c.at[2], left, 0)
        aR = copies(lambda c: c.at[0], lambda c: c.at[1], right, 1)
        bR = copies(lambda c: c.at[1, 0:(HALF_Q if c is commq else HALF_O)],
                    lambda c: c.at[3, 0:(HALF_Q if c is commq else HALF_O)],
                    right, 2)
        bL = copies(lambda c: c.at[2, (HALF_Q if c is commq else HALF_O):],
                    lambda c: c.at[3, (HALF_Q if c is commq else HALF_O):],
                    left, 3)

        if not _SKIP_COMM:
            for c in aL + aR:
                c.start()

        ri = lax.broadcasted_iota(jnp.int32, (SQ, SKV), 0) // BLK
        ci = lax.broadcasted_iota(jnp.int32, (SQ, SKV), 1) // BLK
        mask = (ri == ci) | (ci == 0) | (lax.rem(ri + ci, 3) == 0)
        mask_cat = jnp.tile(mask, (1, H_PER))
        eye_r = lax.broadcasted_iota(jnp.int32, (SKV, SKV), 0)
        eye_c = lax.broadcasted_iota(jnp.int32, (SKV, SKV), 1)
        eye = (eye_r == eye_c).astype(jnp.bfloat16)
        sum_r = lax.broadcasted_iota(jnp.int32, (WCAT, H_PER), 0)
        sum_c = lax.broadcasted_iota(jnp.int32, (WCAT, H_PER), 1)
        ones_bd = (sum_r // SKV == sum_c).astype(jnp.bfloat16)
        ex_r = lax.broadcasted_iota(jnp.int32, (H_PER, WCAT), 0)
        ex_c = lax.broadcasted_iota(jnp.int32, (H_PER, WCAT), 1)
        expand = (ex_c // SKV == ex_r).astype(jnp.bfloat16)

        xb = [x_ref[b].astype(jnp.bfloat16) for b in range(B_LOC)]

        def build_kbd(slot):
            origin = origin_of_slot[slot]
            for b in range(B_LOC):
                kbd[b, slot] = jnp.zeros((CHUNK, WCAT), jnp.bfloat16)
                for j in range(H_PER):
                    hg = origin * H_PER + j
                    kh = k_ref[b, pl.ds(hg, 1)].reshape(SKV, DH)
                    khT = lax.dot_general(
                        kh, eye, (((0,), (0,)), ((), ())),
                        preferred_element_type=jnp.float32
                        ).astype(jnp.bfloat16)
                    kbd[b, slot, j * DH:(j + 1) * DH,
                        j * SKV:(j + 1) * SKV] = khT

        def compute_chunk(slot):
            origin = origin_of_slot[slot]
            wq_c = commq[slot]
            wo_c = commo[slot]
            for b in range(B_LOC):
                qc = jnp.dot(xb[b], wq_c,
                             preferred_element_type=jnp.float32
                             ).astype(jnp.bfloat16)
                s_cat = jnp.dot(qc, kbd[b, slot],
                                preferred_element_type=jnp.float32)
                w_cat = jnp.where(mask_cat,
                                  jnp.exp(s_cat * 0.125),
                                  0.0).astype(jnp.bfloat16)
                sums = jnp.dot(w_cat, ones_bd,
                               preferred_element_type=jnp.float32)
                rw = jnp.dot((1.0 / sums).astype(jnp.bfloat16), expand,
                             preferred_element_type=jnp.float32
                             ).astype(jnp.bfloat16)
                wn = w_cat * rw
                acc = None
                for j in range(H_PER):
                    hg = origin * H_PER + j
                    vh = v_ref[b, pl.ds(hg, 1)].reshape(SKV, DH)
                    ctx_j = jnp.dot(wn[:, j * SKV:(j + 1) * SKV], vh,
                                    preferred_element_type=jnp.float32
                                    ).astype(jnp.bfloat16)
                    contrib = jnp.dot(ctx_j, wo_c[j * DH:(j + 1) * DH],
                                      preferred_element_type=jnp.float32)
                    acc = contrib if acc is None else acc + contrib
                if slot == 0:
                    out_ref[b] = acc
                else:
                    out_ref[b] = out_ref[b] + acc

        build_kbd(0)
        compute_chunk(0)
        for s in range(1, N_DEV):
            build_kbd(s)
        if not _SKIP_COMM:
            aR[0].wait_recv()
            aR[1].wait_recv()
            bR[0].start()
            bR[1].start()
            aL[0].wait_recv()
            aL[1].wait_recv()
            bL[0].start()
            bL[1].start()
        compute_chunk(1)
        compute_chunk(2)
        if not _SKIP_COMM:
            for c in bR + bL:
                c.wait_recv()
        compute_chunk(3)
        if not _SKIP_COMM:
            for c in aL + aR + bR + bL:
                c.wait_send()

    return pl.pallas_call(
        body,
        out_shape=jax.ShapeDtypeStruct((B_LOC, SQ, D_MODEL), jnp.float32),
        in_specs=[pl.BlockSpec(memory_space=pltpu.VMEM)] * 5,
        out_specs=pl.BlockSpec(memory_space=pltpu.VMEM),
        scratch_shapes=[
            pltpu.VMEM((N_DEV, D_MODEL, CHUNK), jnp.bfloat16),
            pltpu.VMEM((N_DEV, CHUNK, D_MODEL), jnp.bfloat16),
            pltpu.VMEM((B_LOC, N_DEV, CHUNK, WCAT), jnp.bfloat16),
            pltpu.SemaphoreType.DMA((4,)),
            pltpu.SemaphoreType.DMA((4,)),
            pltpu.SemaphoreType.DMA((4,)),
            pltpu.SemaphoreType.DMA((4,)),
        ],
        compiler_params=pltpu.CompilerParams(collective_id=0),
    )(x, Wq, k_loc, v_loc, Wo)


# device time: 24421 ns/iter; 1.5621x vs baseline; 1.0690x over previous
import os

import jax
import jax.numpy as jnp
from jax import lax
from jax.experimental import pallas as pl
from jax.experimental.pallas import tpu as pltpu

_SKIP_COMM = os.environ.get("SKIP_COMM", "0") == "1"

N_DEV = 4
B_LOC = 2
SQ = 256
SKV = 256
HQ = 16
DH = 64
D_MODEL = 512
D_HEADS = HQ * DH
CHUNK = D_HEADS // N_DEV
H_PER = HQ // N_DEV
HALF_Q = D_MODEL // 2
HALF_O = CHUNK // 2
BLK = 64


def kernel(x, Wq, K_ext, V_ext, Wo):
    gb0 = lax.axis_index("i") * B_LOC
    k_loc = jnp.transpose(
        lax.dynamic_slice_in_dim(K_ext, gb0, B_LOC, axis=0)
        .astype(jnp.bfloat16), (0, 2, 1, 3))
    v_loc = jnp.transpose(
        lax.dynamic_slice_in_dim(V_ext, gb0, B_LOC, axis=0)
        .astype(jnp.bfloat16), (0, 2, 1, 3))

    def body(x_ref, wq_ref, k_ref, v_ref, wo_ref, out_ref,
             commq, commo, sendq, recvq, sendo, recvo):
        my = lax.axis_index("i")
        left = lax.rem(my + N_DEV - 1, N_DEV)
        right = lax.rem(my + 1, N_DEV)
        opp = lax.rem(my + 2, N_DEV)

        barrier = pltpu.get_barrier_semaphore()
        for nbr in (left, right):
            pl.semaphore_signal(barrier, inc=1, device_id=(nbr,),
                                device_id_type=pl.DeviceIdType.MESH)
        pl.semaphore_wait(barrier, 2)

        commq[0] = wq_ref[...].astype(jnp.bfloat16)
        commo[0] = wo_ref[...].astype(jnp.bfloat16)

        def copies(src, dst, dev, idx):
            cq = pltpu.make_async_remote_copy(
                src_ref=src(commq), dst_ref=dst(commq),
                send_sem=sendq.at[idx], recv_sem=recvq.at[idx],
                device_id=(dev,), device_id_type=pl.DeviceIdType.MESH)
            co = pltpu.make_async_remote_copy(
                src_ref=src(commo), dst_ref=dst(commo),
                send_sem=sendo.at[idx], recv_sem=recvo.at[idx],
                device_id=(dev,), device_id_type=pl.DeviceIdType.MESH)
            return cq, co

        aL = copies(lambda c: c.at[0], lambda c: c.at[2], left, 0)
        aR = copies(lambda c: c.at[0], lambda c: c.at[1], right, 1)
        bR = copies(lambda c: c.at[1, 0:(HALF_Q if c is commq else HALF_O)],
                    lambda c: c.at[3, 0:(HALF_Q if c is commq else HALF_O)],
                    right, 2)
        bL = copies(lambda c: c.at[2, (HALF_Q if c is commq else HALF_O):],
                    lambda c: c.at[3, (HALF_Q if c is commq else HALF_O):],
                    left, 3)

        xb = [x_ref[b].astype(jnp.bfloat16) for b in range(B_LOC)]

        def attend(qh, kh, vh):
            q_o = jnp.concatenate([qh[0:BLK], qh[3 * BLK:]], axis=0)
            k_o = jnp.concatenate([kh[0:BLK], kh[3 * BLK:]], axis=0)
            v_o = jnp.concatenate([vh[0:BLK], vh[3 * BLK:]], axis=0)
            s_o = lax.dot_general(
                q_o, k_o, (((1,), (1,)), ((), ())),
                preferred_element_type=jnp.float32) * 0.125
            w_o = jnp.exp(s_o)
            r_o = 1.0 / jnp.sum(w_o, axis=1, keepdims=True)
            c_o = jnp.dot(w_o.astype(jnp.bfloat16), v_o,
                          preferred_element_type=jnp.float32) * r_o

            q_m = qh[BLK:3 * BLK]
            s_m = lax.dot_general(
                q_m, kh[0:3 * BLK], (((1,), (1,)), ((), ())),
                preferred_element_type=jnp.float32) * 0.125
            w_m = jnp.exp(s_m)
            r_m = 1.0 / jnp.sum(w_m, axis=1, keepdims=True)
            c_m = jnp.dot(w_m.astype(jnp.bfloat16), vh[0:3 * BLK],
                          preferred_element_type=jnp.float32) * r_m
            return jnp.concatenate([c_o[0:BLK], c_m, c_o[BLK:]], axis=0)

        def compute_chunk(slot, origin):
            wq_c = commq[slot]
            wo_c = commo[slot]
            for b in range(B_LOC):
                qc = jnp.dot(xb[b], wq_c,
                             preferred_element_type=jnp.float32)
                ctx_cols = []
                for j in range(H_PER):
                    hg = origin * H_PER + j
                    qh = qc[:, j * DH:(j + 1) * DH].astype(jnp.bfloat16)
                    kh = k_ref[b, pl.ds(hg, 1)].reshape(SKV, DH)
                    vh = v_ref[b, pl.ds(hg, 1)].reshape(SKV, DH)
                    ctx_cols.append(attend(qh, kh, vh))
                ctx = jnp.concatenate(ctx_cols, axis=1).astype(jnp.bfloat16)
                acc = jnp.dot(ctx, wo_c,
                              preferred_element_type=jnp.float32)
                if slot == 0:
                    out_ref[b] = acc
                else:
                    out_ref[b] = out_ref[b] + acc

        if not _SKIP_COMM:
            for c in aL + aR:
                c.start()
        compute_chunk(0, my)
        if not _SKIP_COMM:
            aR[0].wait_recv()
            aR[1].wait_recv()
            bR[0].start()
            bR[1].start()
            aL[0].wait_recv()
            aL[1].wait_recv()
            bL[0].start()
            bL[1].start()
        compute_chunk(1, left)
        compute_chunk(2, right)
        if not _SKIP_COMM:
            for c in bR + bL:
                c.wait_recv()
        compute_chunk(3, opp)
        if not _SKIP_COMM:
            for c in aL + aR + bR + bL:
                c.wait_send()

    return pl.pallas_call(
        body,
        out_shape=jax.ShapeDtypeStruct((B_LOC, SQ, D_MODEL), jnp.float32),
        in_specs=[pl.BlockSpec(memory_space=pltpu.VMEM)] * 5,
        out_specs=pl.BlockSpec(memory_space=pltpu.VMEM),
        scratch_shapes=[
            pltpu.VMEM((N_DEV, D_MODEL, CHUNK), jnp.bfloat16),
            pltpu.VMEM((N_DEV, CHUNK, D_MODEL), jnp.bfloat16),
            pltpu.SemaphoreType.DMA((4,)),
            pltpu.SemaphoreType.DMA((4,)),
            pltpu.SemaphoreType.DMA((4,)),
            pltpu.SemaphoreType.DMA((4,)),
        ],
        compiler_params=pltpu.CompilerParams(collective_id=0),
    )(x, Wq, k_loc, v_loc, Wo)
